# baseline (speedup 1.0000x reference)
import functools

import jax
import jax.numpy as jnp
from jax import lax
from jax.experimental import pallas as pl
from jax.experimental.pallas import tpu as pltpu
from jax.experimental.pallas import tpu_sc as plsc

NC = 2
NS = 16
CH = 128


G = 4


def _sc_aggregate(h_aug, e, z):
    n_pad, da = h_aug.shape
    nw, ngrp = e.shape[0], e.shape[1]
    rpt = n_pad // NS

    mesh = plsc.VectorSubcoreMesh(core_axis_name="c", subcore_axis_name="s")

    @functools.partial(
        pl.kernel,
        out_type=jax.ShapeDtypeStruct((NC, n_pad, da), jnp.float32),
        mesh=mesh,
        scratch_types=[
            pltpu.VMEM((2, 2, G, CH), jnp.int32),
            pltpu.VMEM((2, CH, da), jnp.float32),
            pltpu.VMEM_SHARED((n_pad, da), jnp.float32),
            pltpu.SemaphoreType.DMA((2,)),
            pltpu.SemaphoreType.DMA((2,)),
        ],
        compiler_params=pltpu.CompilerParams(use_tc_tiling_on_sc=False),
    )
    def sc_k(haug_hbm, e_hbm, z_hbm, out_hbm, idx_v, rows_v, acc_sh, gsem, isem):
        c = lax.axis_index("c")
        s = lax.axis_index("s")
        wid = c * NS + s
        r0 = s * rpt
        last = ngrp - 1

        def idx_dma(g, ib):
            return pltpu.make_async_copy(e_hbm.at[wid, g], idx_v.at[ib],
                                         isem.at[ib])

        def gather(ib, j, rb):
            return pltpu.make_async_copy(haug_hbm.at[idx_v.at[ib, 0, j]],
                                         rows_v.at[rb], gsem.at[rb])

        idx_dma(0, 0).start()
        idx_dma(jnp.minimum(1, last), 1).start()
        pltpu.sync_copy(z_hbm.at[pl.ds(r0, rpt)], acc_sh.at[pl.ds(r0, rpt)])
        plsc.subcore_barrier()

        idx_dma(0, 0).wait()
        gather(0, 0, 0).start()
        gather(0, 1, 1).start()

        @pl.loop(0, ngrp, step=2)
        def _(g):
            for gb in range(2):
                cg = g + gb
                for j in range(G):
                    rb = j % 2
                    gather(gb, j, rb).wait()
                    pltpu.sync_copy(rows_v.at[rb],
                                    acc_sh.at[idx_v.at[gb, 1, j]], add=True)
                    if j == G - 2:
                        idx_dma(0, gb ^ 1).wait()
                    if j < G - 2:
                        gather(gb, j + 2, rb).start()
                    else:
                        gather(gb ^ 1, j + 2 - G, rb).start()
                    if j == G - 1:
                        idx_dma(jnp.minimum(cg + 2, last), gb).start()

        idx_dma(0, 1).wait()
        gather(0, 0, 0).wait()
        gather(0, 1, 1).wait()

        plsc.subcore_barrier()
        pltpu.sync_copy(acc_sh.at[pl.ds(r0, rpt)], out_hbm.at[c, pl.ds(r0, rpt)])

    return sc_k(h_aug, e, z)


def _tc_finish(acc, h, w, b2):
    n, d = h.shape
    da = acc.shape[2]
    d_out = w.shape[1]
    blk = 1000 if n % 1000 == 0 else 8
    grid = n // blk

    def body(acc_ref, h_ref, w_ref, b_ref, o_ref):
        p = acc_ref[0] + acc_ref[1]
        sums = p[:, :d]
        cnt = p[:, d:d + 1]
        h_n = sums / jnp.maximum(cnt, 1.0)
        o_ref[...] = (
            jnp.dot(h_ref[...], w_ref[:d, :], preferred_element_type=jnp.float32)
            + jnp.dot(h_n, w_ref[d:, :], preferred_element_type=jnp.float32)
            + b_ref[...]
        )

    return pl.pallas_call(
        body,
        grid=(grid,),
        in_specs=[
            pl.BlockSpec((2, blk, da), lambda i: (0, i, 0)),
            pl.BlockSpec((blk, d), lambda i: (i, 0)),
            pl.BlockSpec((2 * d, d_out), lambda i: (0, 0)),
            pl.BlockSpec((1, d_out), lambda i: (0, 0)),
        ],
        out_specs=pl.BlockSpec((blk, d_out), lambda i: (i, 0)),
        out_shape=jax.ShapeDtypeStruct((n, d_out), jnp.float32),
    )(acc, h, w, b2)


def kernel(h, edge_index, W, b):
    n, d = h.shape
    e_cnt = edge_index.shape[1]
    da = ((d + 1 + 15) // 16) * 16
    n_pad = ((n + 1 + NS * 8 - 1) // (NS * 8)) * (NS * 8)
    nw = NC * NS
    step = nw * CH * G * 2
    e_pad_cnt = ((e_cnt + step - 1) // step) * step
    ngrp = e_pad_cnt // (nw * CH * G)

    e32 = edge_index.astype(jnp.int32)
    if e_pad_cnt != e_cnt:
        npe = e_pad_cnt - e_cnt
        pad_dst = n + jnp.arange(npe, dtype=jnp.int32) % (n_pad - n)
        pad = jnp.stack([jnp.full((npe,), n, jnp.int32), pad_dst])
        e32 = jnp.concatenate([e32, pad], axis=1)
    e32 = e32.reshape(2, nw, ngrp, G, CH).transpose(1, 2, 0, 3, 4)
    h_aug = jnp.zeros((n_pad, da), jnp.float32)
    h_aug = h_aug.at[:n, :d].set(h).at[:n, d].set(1.0)
    z = jnp.zeros((n_pad, da), jnp.float32)

    acc = _sc_aggregate(h_aug, e32, z)
    return _tc_finish(acc[:, :n, :], h, W, b.reshape(1, -1))


# speedup vs baseline: 2.5957x; 2.5957x over previous
import functools

import jax
import jax.numpy as jnp
from jax import lax
from jax.experimental import pallas as pl
from jax.experimental.pallas import tpu as pltpu
from jax.experimental.pallas import tpu_sc as plsc

NC = 2
NS = 16
CH = 128
G = 4


def _sc_aggregate(h2, e, z):
    _, n_pad, dh = h2.shape
    ngrp = e.shape[1]
    rpt = n_pad // NS

    mesh = plsc.VectorSubcoreMesh(core_axis_name="c", subcore_axis_name="s")

    @functools.partial(
        pl.kernel,
        out_type=jax.ShapeDtypeStruct((NC, n_pad, dh), jnp.float32),
        mesh=mesh,
        scratch_types=[
            pltpu.VMEM((2, 2, G, CH), jnp.int32),
            pltpu.VMEM((2, CH, dh), jnp.float32),
            pltpu.VMEM_SHARED((n_pad, dh), jnp.float32),
            pltpu.VMEM_SHARED((n_pad, dh), jnp.float32),
            pltpu.SemaphoreType.DMA((2,)),
            pltpu.SemaphoreType.DMA((2,)),
        ],
        compiler_params=pltpu.CompilerParams(use_tc_tiling_on_sc=False),
    )
    def sc_k(h2_hbm, e_hbm, z_hbm, out_hbm, idx_v, rows_v, tab_sh, acc_sh,
             gsem, isem):
        c = lax.axis_index("c")
        s = lax.axis_index("s")
        r0 = s * rpt
        last = ngrp - 1

        def idx_dma(g, ib):
            return pltpu.make_async_copy(e_hbm.at[s, g], idx_v.at[ib],
                                         isem.at[ib])

        def gather(ib, j, rb):
            return pltpu.make_async_copy(tab_sh.at[idx_v.at[ib, 0, j]],
                                         rows_v.at[rb], gsem.at[rb])

        idx_dma(0, 0).start()
        idx_dma(jnp.minimum(1, last), 1).start()
        pltpu.sync_copy(h2_hbm.at[c, pl.ds(r0, rpt)], tab_sh.at[pl.ds(r0, rpt)])
        pltpu.sync_copy(z_hbm.at[pl.ds(r0, rpt)], acc_sh.at[pl.ds(r0, rpt)])
        plsc.subcore_barrier()

        idx_dma(0, 0).wait()
        gather(0, 0, 0).start()
        gather(0, 1, 1).start()

        @pl.loop(0, ngrp, step=2)
        def _(g):
            for gb in range(2):
                cg = g + gb
                for j in range(G):
                    rb = j % 2
                    gather(gb, j, rb).wait()
                    pltpu.sync_copy(rows_v.at[rb],
                                    acc_sh.at[idx_v.at[gb, 1, j]], add=True)
                    if j == G - 2:
                        idx_dma(0, gb ^ 1).wait()
                    if j < G - 2:
                        gather(gb, j + 2, rb).start()
                    else:
                        gather(gb ^ 1, j + 2 - G, rb).start()
                    if j == G - 1:
                        idx_dma(jnp.minimum(cg + 2, last), gb).start()

        idx_dma(0, 1).wait()
        gather(0, 0, 0).wait()
        gather(0, 1, 1).wait()

        plsc.subcore_barrier()
        pltpu.sync_copy(acc_sh.at[pl.ds(r0, rpt)], out_hbm.at[c, pl.ds(r0, rpt)])

    return sc_k(h2, e, z)


def _tc_finish(acc, h, w_self, wn2, b2, cnt_col):
    n, d = h.shape
    dh = acc.shape[2]
    d_out = w_self.shape[1]
    blk = 1000 if n % 1000 == 0 else 8
    grid = n // blk

    def body(acc_ref, h_ref, ws_ref, wn_ref, b_ref, o_ref):
        p0 = acc_ref[0]
        p1 = acc_ref[1]
        cnt = p1[:, cnt_col:cnt_col + 1]
        neigh = (
            jnp.dot(p0, wn_ref[0], preferred_element_type=jnp.float32)
            + jnp.dot(p1, wn_ref[1], preferred_element_type=jnp.float32)
        ) / jnp.maximum(cnt, 1.0)
        o_ref[...] = (
            jnp.dot(h_ref[...], ws_ref[...], preferred_element_type=jnp.float32)
            + neigh + b_ref[...]
        )

    return pl.pallas_call(
        body,
        grid=(grid,),
        in_specs=[
            pl.BlockSpec((2, blk, dh), lambda i: (0, i, 0)),
            pl.BlockSpec((blk, d), lambda i: (i, 0)),
            pl.BlockSpec((d, d_out), lambda i: (0, 0)),
            pl.BlockSpec((2, dh, d_out), lambda i: (0, 0, 0)),
            pl.BlockSpec((1, d_out), lambda i: (0, 0)),
        ],
        out_specs=pl.BlockSpec((blk, d_out), lambda i: (i, 0)),
        out_shape=jax.ShapeDtypeStruct((n, d_out), jnp.float32),
    )(acc, h, w_self, wn2, b2)


def kernel(h, edge_index, W, b):
    n, d = h.shape
    e_cnt = edge_index.shape[1]
    da = ((d + 1 + 31) // 32) * 32
    dh = da // 2
    n_pad = ((n + 1 + NS * 8 - 1) // (NS * 8)) * (NS * 8)
    step = NS * CH * G * 2
    e_pad_cnt = ((e_cnt + step - 1) // step) * step
    ngrp = e_pad_cnt // (NS * CH * G)

    e32 = edge_index.astype(jnp.int32)
    if e_pad_cnt != e_cnt:
        npe = e_pad_cnt - e_cnt
        pad_dst = n + jnp.arange(npe, dtype=jnp.int32) % (n_pad - n)
        pad = jnp.stack([jnp.full((npe,), n, jnp.int32), pad_dst])
        e32 = jnp.concatenate([e32, pad], axis=1)
    e32 = e32.reshape(2, NS, ngrp, G, CH).transpose(1, 2, 0, 3, 4)

    h_aug = jnp.zeros((n_pad, da), jnp.float32)
    h_aug = h_aug.at[:n, :d].set(h).at[:n, d].set(1.0)
    h2 = h_aug.reshape(n_pad, 2, dh).transpose(1, 0, 2)
    z = jnp.zeros((n_pad, dh), jnp.float32)

    acc = _sc_aggregate(h2, e32, z)

    wn = W[d:]
    wn2 = jnp.zeros((2, dh, W.shape[1]), jnp.float32)
    wn2 = wn2.at[0].set(wn[:dh]).at[1, :d - dh].set(wn[dh:])
    cnt_col = d - dh
    return _tc_finish(acc[:, :n, :], h, W[:d], wn2, b.reshape(1, -1), cnt_col)
